# baseline (speedup 1.0000x reference)
import jax
import jax.numpy as jnp
from jax.experimental import pallas as pl
from jax.experimental.pallas import tpu as pltpu

_SEQ = 8192
_W = 32
_DIM = _W * _W
_NDMA = 16
_ROWS = _SEQ // _NDMA


def _tc_body(in_hbm, out_hbm, sems):
    for i in range(_NDMA):
        pltpu.make_async_copy(
            in_hbm.at[pl.ds(i * _ROWS, _ROWS)],
            out_hbm.at[pl.ds(i * _ROWS, _ROWS)],
            sems.at[i],
        ).start()
    for i in range(_NDMA):
        pltpu.make_async_copy(
            in_hbm.at[pl.ds(i * _ROWS, _ROWS)],
            out_hbm.at[pl.ds(i * _ROWS, _ROWS)],
            sems.at[i],
        ).wait()


def kernel(x, position_embeddings):
    del x
    out = pl.pallas_call(
        _tc_body,
        in_specs=[pl.BlockSpec(memory_space=pltpu.MemorySpace.HBM)],
        out_specs=pl.BlockSpec(memory_space=pltpu.MemorySpace.HBM),
        out_shape=jax.ShapeDtypeStruct((_SEQ, _DIM), jnp.float32),
        scratch_shapes=[pltpu.SemaphoreType.DMA((_NDMA,))],
    )(position_embeddings)
    return out.reshape(1, _SEQ, _W, _W)


# speedup vs baseline: 14.1537x; 14.1537x over previous
import functools

import jax
import jax.numpy as jnp
from jax import lax
from jax.experimental import pallas as pl
from jax.experimental.pallas import tpu as pltpu
from jax.experimental.pallas import tpu_sc as plsc

_SEQ = 8192
_W = 32
_DIM = _W * _W

_SPLIT = 4096

_NBUF = 3
_CHUNK = 32

_TC_BLK = 1024


def _sc_body(table_hbm, out_hbm, *scratch):
    bufs = scratch[:_NBUF]
    sin = scratch[_NBUF:2 * _NBUF]
    sout = scratch[2 * _NBUF:]
    info = plsc.get_sparse_core_info()
    nw = info.num_cores * info.num_subcores
    rows = _SPLIT // nw
    nchunks = rows // _CHUNK
    wid = lax.axis_index("s") * info.num_cores + lax.axis_index("c")
    base = wid * rows

    def in_copy(b, c):
        return pltpu.make_async_copy(
            table_hbm.at[pl.ds(base + c * _CHUNK, _CHUNK)], bufs[b], sin[b])

    def out_copy(b, c):
        return pltpu.make_async_copy(
            bufs[b], out_hbm.at[pl.ds(base + c * _CHUNK, _CHUNK)], sout[b])

    for b in range(min(_NBUF, nchunks)):
        in_copy(b, b).start()
    for c in range(nchunks):
        b = c % _NBUF
        in_copy(b, c).wait()
        out_copy(b, c).start()
        nxt = c + _NBUF
        if nxt < nchunks:
            out_copy(b, c).wait()
            in_copy(b, nxt).start()
    for c in range(max(0, nchunks - _NBUF), nchunks):
        out_copy(c % _NBUF, c).wait()


def _tc_body(table_blk, partial_hbm, out_blk):
    del partial_hbm
    out_blk[...] = table_blk[...]


def kernel(x, position_embeddings):
    del x
    mesh = plsc.VectorSubcoreMesh(core_axis_name="c", subcore_axis_name="s")
    sc_copy = functools.partial(
        pl.kernel,
        mesh=mesh,
        out_type=jax.ShapeDtypeStruct((_SEQ, _DIM), jnp.float32),
        scratch_types=(
            [pltpu.VMEM((_CHUNK, _DIM), jnp.float32) for _ in range(_NBUF)]
            + [pltpu.SemaphoreType.DMA for _ in range(2 * _NBUF)]
        ),
    )(_sc_body)
    partial = sc_copy(position_embeddings)

    nblk = (_SEQ - _SPLIT) // _TC_BLK
    base_blk = _SPLIT // _TC_BLK
    out = pl.pallas_call(
        _tc_body,
        grid=(nblk,),
        in_specs=[
            pl.BlockSpec((_TC_BLK, _DIM), lambda i: (base_blk + i, 0)),
            pl.BlockSpec(memory_space=pltpu.MemorySpace.HBM),
        ],
        out_specs=pl.BlockSpec((_TC_BLK, _DIM), lambda i: (base_blk + i, 0)),
        out_shape=jax.ShapeDtypeStruct((_SEQ, _DIM), jnp.float32),
        input_output_aliases={1: 0},
    )(position_embeddings, partial)
    return out.reshape(1, _SEQ, _W, _W)


# speedup vs baseline: 14.5012x; 1.0246x over previous
import functools

import jax
import jax.numpy as jnp
from jax import lax
from jax.experimental import pallas as pl
from jax.experimental.pallas import tpu as pltpu
from jax.experimental.pallas import tpu_sc as plsc

_SEQ = 8192
_W = 32
_DIM = _W * _W

_NBUF = 3
_CHUNK = 32


def _sc_body(table_hbm, out_hbm, *scratch):
    bufs = scratch[:_NBUF]
    sin = scratch[_NBUF:2 * _NBUF]
    sout = scratch[2 * _NBUF:]
    info = plsc.get_sparse_core_info()
    nw = info.num_cores * info.num_subcores
    rows = _SEQ // nw
    nchunks = rows // _CHUNK
    wid = lax.axis_index("s") * info.num_cores + lax.axis_index("c")
    base = wid * rows

    def in_copy(b, c):
        return pltpu.make_async_copy(
            table_hbm.at[pl.ds(base + c * _CHUNK, _CHUNK)], bufs[b], sin[b])

    def out_copy(b, c):
        return pltpu.make_async_copy(
            bufs[b], out_hbm.at[pl.ds(base + c * _CHUNK, _CHUNK)], sout[b])

    for b in range(min(_NBUF, nchunks)):
        in_copy(b, b).start()
    for c in range(nchunks):
        b = c % _NBUF
        in_copy(b, c).wait()
        out_copy(b, c).start()
        nxt = c + _NBUF
        if nxt < nchunks:
            out_copy(b, c).wait()
            in_copy(b, nxt).start()
    for c in range(max(0, nchunks - _NBUF), nchunks):
        out_copy(c % _NBUF, c).wait()


def kernel(x, position_embeddings):
    del x
    mesh = plsc.VectorSubcoreMesh(core_axis_name="c", subcore_axis_name="s")
    sc_copy = functools.partial(
        pl.kernel,
        mesh=mesh,
        out_type=jax.ShapeDtypeStruct((_SEQ, _DIM), jnp.float32),
        scratch_types=(
            [pltpu.VMEM((_CHUNK, _DIM), jnp.float32) for _ in range(_NBUF)]
            + [pltpu.SemaphoreType.DMA for _ in range(2 * _NBUF)]
        ),
    )(_sc_body)
    out = sc_copy(position_embeddings)
    return out.reshape(1, _SEQ, _W, _W)
